# baseline (speedup 1.0000x reference)
import functools

import jax
import jax.numpy as jnp
from jax import lax
from jax.experimental import pallas as pl
from jax.experimental.pallas import tpu as pltpu
from jax.experimental.pallas import tpu_sc as plsc

N = 10000
E = 160000
C = 256
EPS = 1e-05

NT = 16
K = 128
NPH = 2
PCH = 40
NCHUNK = NPH * PCH
NPAIR = PCH // 2
EPT = NCHUNK * K
EP = NT * EPT
RW = 128
NROWS = 10112
RPT = NROWS // NT
CROWS = 80
NROWS2 = NROWS + CROWS
BN = 2000
GRID = N // BN


@functools.lru_cache(maxsize=None)
def _make_sc_kernel(with_cnt):
    mesh = plsc.VectorSubcoreMesh(core_axis_name="c", subcore_axis_name="s")
    nr = NROWS2 if with_cnt else NROWS
    out_type = jax.ShapeDtypeStruct((2 * nr, RW), jnp.float32)
    scratch = [
        pltpu.VMEM((PCH, K), jnp.int32),
        pltpu.VMEM((PCH, K), jnp.int32),
        pltpu.VMEM((K, RW), jnp.float32),
        pltpu.VMEM((K, RW), jnp.float32),
        pltpu.VMEM_SHARED((nr, RW), jnp.float32),
        pltpu.SemaphoreType.DMA,
        pltpu.SemaphoreType.DMA,
    ]
    if with_cnt:
        scratch += [
            pltpu.VMEM((1, K), jnp.int32),
            pltpu.VMEM((1, K), jnp.int32),
        ]

    @functools.partial(pl.kernel, out_type=out_type, mesh=mesh,
                       scratch_types=scratch)
    def k(fx_hbm, src_hbm, dst_hbm, z_hbm, eye_hbm, *rest):
        if with_cnt:
            out_hbm, src_v, dst_v, rows_a, rows_b, s_sh, sem_a, sem_b, lo_v, hi_v = rest
        else:
            out_hbm, src_v, dst_v, rows_a, rows_b, s_sh, sem_a, sem_b = rest
        c = lax.axis_index("c")
        w = lax.axis_index("s")
        coff = c * N

        pltpu.sync_copy(z_hbm.at[pl.ds(w * RPT, RPT)], s_sh.at[pl.ds(w * RPT, RPT)])
        if with_cnt:
            @pl.when(w == 0)
            def _():
                pltpu.sync_copy(z_hbm.at[pl.ds(0, CROWS)],
                                s_sh.at[pl.ds(NROWS, CROWS)])
        plsc.subcore_barrier()

        for ph in range(NPH):
            pltpu.sync_copy(src_hbm.at[w, ph], src_v)
            pltpu.sync_copy(dst_hbm.at[w, ph], dst_v)

            def addoff(j, carry):
                for t in range(K // 16):
                    sl = pl.ds(t * 16, 16)
                    src_v[j, sl] = src_v[j, sl] + coff
                return carry

            lax.fori_loop(0, PCH, addoff, 0)

            pltpu.async_copy(fx_hbm.at[src_v.at[0]], rows_a, sem_a)
            pltpu.async_copy(fx_hbm.at[src_v.at[1]], rows_b, sem_b)

            def halfstep(j, rows_v, sem):
                pltpu.make_async_copy(fx_hbm.at[src_v.at[j]], rows_v, sem).wait()
                pltpu.sync_copy(rows_v, s_sh.at[dst_v.at[j]], add=True)
                if with_cnt:
                    @pl.when(lax.bitwise_and(j, 1) == c)
                    def _():
                        for t in range(K // 16):
                            sl = pl.ds(t * 16, 16)
                            d16 = dst_v[j, sl]
                            lo_v[0, sl] = lax.bitwise_and(d16, 127)
                            hi_v[0, sl] = lax.shift_right_logical(d16, 7) + NROWS
                        pltpu.sync_copy(eye_hbm.at[lo_v.at[0]], rows_v)
                        pltpu.sync_copy(rows_v, s_sh.at[hi_v.at[0]], add=True)

                @pl.when(j + 2 < PCH)
                def _():
                    pltpu.async_copy(fx_hbm.at[src_v.at[j + 2]], rows_v, sem)

            def pair(jo, carry):
                j0 = 2 * jo
                halfstep(j0, rows_a, sem_a)
                halfstep(j0 + 1, rows_b, sem_b)
                return carry

            lax.fori_loop(0, NPAIR, pair, 0)
        plsc.subcore_barrier()

        pltpu.sync_copy(s_sh.at[pl.ds(w * RPT, RPT)],
                        out_hbm.at[pl.ds(c * nr + w * RPT, RPT)])
        if with_cnt:
            @pl.when(w == 0)
            def _():
                pltpu.sync_copy(s_sh.at[pl.ds(NROWS, CROWS)],
                                out_hbm.at[pl.ds(c * nr + NROWS, CROWS)])

    return k


def _sc_segment_sum(fxcat, srcidx, dstidx, zrows, eye, with_cnt):
    res = _make_sc_kernel(with_cnt)(fxcat, srcidx, dstidx, zrows, eye)
    return res[0] if isinstance(res, (list, tuple)) else res


def _powmsg(xmsg, p):
    cl = jnp.clip(xmsg, 0.0, 100.0)
    gen = jnp.exp(p * jnp.log(jnp.maximum(cl, 1e-30)))
    return jnp.where(p == 1.0, cl, gen)


def _prep_body(p_ref, x_ref, fx_ref):
    p = p_ref[0, 0]
    msg = jax.nn.relu(x_ref[...]) + EPS
    fx = _powmsg(msg, p)
    fx_ref[0] = fx[:, :RW]
    fx_ref[1] = fx[:, RW:]


def _prep(p, x):
    return pl.pallas_call(
        _prep_body,
        grid=(GRID,),
        in_specs=[
            pl.BlockSpec((1, 1), lambda i: (0, 0)),
            pl.BlockSpec((BN, C), lambda i: (i, 0)),
        ],
        out_specs=pl.BlockSpec((2, BN, RW), lambda i: (0, i, 0)),
        out_shape=jax.ShapeDtypeStruct((2, N, RW), jnp.float32),
    )(p, x)


def _mid_body(p_ref, xin_ref, sa_ref, sb_ref, cnta_ref, cntb_ref, w1_ref, b1_ref,
              h1_ref, sum_ref, ssq_ref, *, first):
    i = pl.program_id(0)
    p = p_ref[0, 0]
    xin = xin_ref[...]
    if not first:
        xin = jax.nn.relu(xin) + EPS
    s = jnp.concatenate([sa_ref[0], sb_ref[0]], axis=1)
    agg = s / jnp.maximum(cnta_ref[...] + cntb_ref[...], 1.0)
    out = _powmsg(agg, 1.0 / p)
    nrm = jnp.sqrt(jnp.sum(out * out, axis=1, keepdims=True))
    out = out / jnp.maximum(nrm, 1e-12)
    xnrm = jnp.sqrt(jnp.sum(xin * xin, axis=1, keepdims=True))
    out = out * xnrm + xin
    h1 = lax.dot_general(out, w1_ref[...], (((1,), (0,)), ((), ())),
                         preferred_element_type=jnp.float32) + b1_ref[...]
    h1_ref[...] = h1

    @pl.when(i == 0)
    def _():
        sum_ref[...] = jnp.zeros_like(sum_ref)
        ssq_ref[...] = jnp.zeros_like(ssq_ref)

    sum_ref[...] += jnp.sum(h1, axis=0, keepdims=True)
    ssq_ref[...] += jnp.sum(h1 * h1, axis=0, keepdims=True)


def _mid(p, xin, s2, cnta, cntb, w1, b1, first):
    return pl.pallas_call(
        functools.partial(_mid_body, first=first),
        grid=(GRID,),
        in_specs=[
            pl.BlockSpec((1, 1), lambda i: (0, 0)),
            pl.BlockSpec((BN, C), lambda i: (i, 0)),
            pl.BlockSpec((1, BN, RW), lambda i: (0, i, 0)),
            pl.BlockSpec((1, BN, RW), lambda i: (1, i, 0)),
            pl.BlockSpec((BN, 1), lambda i: (i, 0)),
            pl.BlockSpec((BN, 1), lambda i: (i, 0)),
            pl.BlockSpec((C, C), lambda i: (0, 0)),
            pl.BlockSpec((1, C), lambda i: (0, 0)),
        ],
        out_specs=[
            pl.BlockSpec((BN, C), lambda i: (i, 0)),
            pl.BlockSpec((1, C), lambda i: (0, 0)),
            pl.BlockSpec((1, C), lambda i: (0, 0)),
        ],
        out_shape=[
            jax.ShapeDtypeStruct((N, C), jnp.float32),
            jax.ShapeDtypeStruct((1, C), jnp.float32),
            jax.ShapeDtypeStruct((1, C), jnp.float32),
        ],
    )(p, xin, s2, s2, cnta, cntb, w1, b1)


def _bn_relu(h1, sum_, ssq, g, be):
    mu = sum_ * (1.0 / N)
    var = ssq * (1.0 / N) - mu * mu
    inv = lax.rsqrt(var + 1e-05)
    return jax.nn.relu((h1 - mu) * inv * g + be)


def _post_prep_body(h1_ref, sum_ref, ssq_ref, g_ref, be_ref, w2_ref, b2_ref,
                    pn_ref, c0_ref, fx_ref):
    h = _bn_relu(h1_ref[...], sum_ref[...], ssq_ref[...], g_ref[...], be_ref[...])
    c0 = lax.dot_general(h, w2_ref[...], (((1,), (0,)), ((), ())),
                         preferred_element_type=jnp.float32) + b2_ref[...]
    c0_ref[...] = c0
    pn = pn_ref[0, 0]
    msg = jax.nn.relu(c0) + 2.0 * EPS
    fx = _powmsg(msg, pn)
    fx_ref[0] = fx[:, :RW]
    fx_ref[1] = fx[:, RW:]


def _post_prep(h1, sum_, ssq, g, be, w2, b2, pn):
    return pl.pallas_call(
        _post_prep_body,
        grid=(GRID,),
        in_specs=[
            pl.BlockSpec((BN, C), lambda i: (i, 0)),
            pl.BlockSpec((1, C), lambda i: (0, 0)),
            pl.BlockSpec((1, C), lambda i: (0, 0)),
            pl.BlockSpec((1, C), lambda i: (0, 0)),
            pl.BlockSpec((1, C), lambda i: (0, 0)),
            pl.BlockSpec((C, C), lambda i: (0, 0)),
            pl.BlockSpec((1, C), lambda i: (0, 0)),
            pl.BlockSpec((1, 1), lambda i: (0, 0)),
        ],
        out_specs=[
            pl.BlockSpec((BN, C), lambda i: (i, 0)),
            pl.BlockSpec((2, BN, RW), lambda i: (0, i, 0)),
        ],
        out_shape=[
            jax.ShapeDtypeStruct((N, C), jnp.float32),
            jax.ShapeDtypeStruct((2, N, RW), jnp.float32),
        ],
    )(h1, sum_, ssq, g, be, w2, b2, pn)


def _post_final_body(h1_ref, sum_ref, ssq_ref, g_ref, be_ref, w2_ref, b2_ref,
                     h0_ref, we_ref, bexp_ref, y_ref):
    h = _bn_relu(h1_ref[...], sum_ref[...], ssq_ref[...], g_ref[...], be_ref[...])
    c1 = lax.dot_general(h, w2_ref[...], (((1,), (0,)), ((), ())),
                         preferred_element_type=jnp.float32) + b2_ref[...]
    t = h0_ref[...] + c1
    y = lax.dot_general(t, we_ref[...], (((1,), (0,)), ((), ())),
                        preferred_element_type=jnp.float32) + bexp_ref[...]
    y_ref[...] = jax.nn.relu(y) + EPS


def _post_final(h1, sum_, ssq, g, be, w2, b2, h0, we, bexp):
    return pl.pallas_call(
        _post_final_body,
        grid=(GRID,),
        in_specs=[
            pl.BlockSpec((BN, C), lambda i: (i, 0)),
            pl.BlockSpec((1, C), lambda i: (0, 0)),
            pl.BlockSpec((1, C), lambda i: (0, 0)),
            pl.BlockSpec((1, C), lambda i: (0, 0)),
            pl.BlockSpec((1, C), lambda i: (0, 0)),
            pl.BlockSpec((C, C), lambda i: (0, 0)),
            pl.BlockSpec((1, C), lambda i: (0, 0)),
            pl.BlockSpec((BN, C), lambda i: (i, 0)),
            pl.BlockSpec((C, 2 * C), lambda i: (0, 0)),
            pl.BlockSpec((1, 2 * C), lambda i: (0, 0)),
        ],
        out_specs=pl.BlockSpec((BN, 2 * C), lambda i: (i, 0)),
        out_shape=jax.ShapeDtypeStruct((N, 2 * C), jnp.float32),
    )(h1, sum_, ssq, g, be, w2, b2, h0, we, bexp)


def kernel(x, edge_index, p0, W1_0, b1_0, g_0, be_0, W2_0, b2_0,
           p1, W1_1, b1_1, g_1, be_1, W2_1, b2_1, We, bexp):
    src = edge_index[0]
    dst = edge_index[1]
    pad = EP - E
    srcp = jnp.concatenate([src, jnp.zeros((pad,), jnp.int32)]).reshape(NT, NPH, PCH, K)
    dstp = jnp.concatenate([dst, jnp.full((pad,), N, jnp.int32)]).reshape(NT, NPH, PCH, K)
    zrows = jnp.zeros((NROWS, RW), jnp.float32)
    eye = jnp.eye(RW, dtype=jnp.float32)
    p0r = p0.reshape(1, 1)
    p1r = p1.reshape(1, 1)

    fx0 = _prep(p0r, x)
    s0 = _sc_segment_sum(fx0.reshape(2 * N, RW), srcp, dstp, zrows, eye, True)
    cnta = s0[NROWS:NROWS2].reshape(CROWS * RW)[:N].reshape(N, 1)
    cntb = s0[NROWS2 + NROWS:].reshape(CROWS * RW)[:N].reshape(N, 1)
    s0 = s0.reshape(2, NROWS2, RW)
    h1_0, sm0, sq0 = _mid(p0r, x, s0, cnta, cntb, W1_0, b1_0.reshape(1, C),
                          first=True)
    c0, fx1 = _post_prep(h1_0, sm0, sq0, g_0.reshape(1, C), be_0.reshape(1, C),
                         W2_0, b2_0.reshape(1, C), p1r)
    s1 = _sc_segment_sum(fx1.reshape(2 * N, RW), srcp, dstp, zrows, eye, False)
    s1 = s1.reshape(2, NROWS, RW)
    h1_1, sm1, sq1 = _mid(p1r, c0, s1, cnta, cntb, W1_1, b1_1.reshape(1, C),
                          first=False)
    return _post_final(h1_1, sm1, sq1, g_1.reshape(1, C), be_1.reshape(1, C),
                       W2_1, b2_1.reshape(1, C), x, We, bexp.reshape(1, 2 * C))


# speedup vs baseline: 1.1930x; 1.1930x over previous
import functools

import jax
import jax.numpy as jnp
from jax import lax
from jax.experimental import pallas as pl
from jax.experimental.pallas import tpu as pltpu
from jax.experimental.pallas import tpu_sc as plsc

N = 10000
E = 160000
C = 256
EPS = 1e-05

NT = 16
K = 128
NCHUNK = 79
EPT = NCHUNK * K
EP = NT * EPT
RW = 128
NROWS = 10112
RPT = NROWS // NT
HB = 80
BN = 2000
GRID = N // BN


@functools.lru_cache(maxsize=None)
def _make_sc_kernel(with_cnt):
    mesh = plsc.VectorSubcoreMesh(core_axis_name="c", subcore_axis_name="s")
    out_type = [jax.ShapeDtypeStruct((2 * NROWS, RW), jnp.float32)]
    scratch = [
        pltpu.VMEM((NCHUNK, K), jnp.int32),
        pltpu.VMEM((NCHUNK, K), jnp.int32),
        pltpu.VMEM((K, RW), jnp.float32),
        pltpu.VMEM_SHARED((NROWS, RW), jnp.float32),
    ]
    if with_cnt:
        out_type.append(jax.ShapeDtypeStruct((2 * HB, RW), jnp.float32))
        scratch += [
            pltpu.VMEM((HB, RW), jnp.float32),
            pltpu.VMEM_SHARED((HB, RW), jnp.float32),
            pltpu.VMEM((1, HB), jnp.int32),
            pltpu.SMEM((K,), jnp.int32),
        ]

    @functools.partial(pl.kernel, out_type=out_type, mesh=mesh,
                       scratch_types=scratch)
    def k(fx_hbm, src_hbm, dst_hbm, z_hbm, iota_hbm, *rest):
        if with_cnt:
            (out_hbm, cnt_hbm, src_v, dst_v, rows_v, s_sh,
             hist_v, cnt_sh, iota_v, dsm) = rest
        else:
            out_hbm, src_v, dst_v, rows_v, s_sh = rest
        c = lax.axis_index("c")
        w = lax.axis_index("s")
        coff = c * N

        pltpu.sync_copy(z_hbm.at[pl.ds(w * RPT, RPT)], s_sh.at[pl.ds(w * RPT, RPT)])
        if with_cnt:
            @pl.when(w == 0)
            def _():
                pltpu.sync_copy(z_hbm.at[pl.ds(0, HB)], cnt_sh)

            zero16 = jnp.zeros((16,), jnp.float32)

            def zhist(r, carry):
                for t in range(RW // 16):
                    hist_v[r, pl.ds(t * 16, 16)] = zero16
                return carry

            lax.fori_loop(0, HB, zhist, 0)
        plsc.subcore_barrier()

        pltpu.sync_copy(src_hbm.at[w], src_v)
        pltpu.sync_copy(dst_hbm.at[w], dst_v)

        def addoff(j, carry):
            for t in range(K // 16):
                sl = pl.ds(t * 16, 16)
                src_v[j, sl] = src_v[j, sl] + coff
            return carry

        lax.fori_loop(0, NCHUNK, addoff, 0)

        iota16 = lax.iota(jnp.int32, 16)
        one16 = jnp.ones((16,), jnp.float32)
        zero16f = jnp.zeros((16,), jnp.float32)

        def body(j, carry):
            pltpu.sync_copy(fx_hbm.at[src_v.at[j]], rows_v)
            pltpu.sync_copy(rows_v, s_sh.at[dst_v.at[j]], add=True)
            if with_cnt:
                @pl.when(lax.bitwise_and(j, 1) == c)
                def _():
                    def cgroup(t, carry2):
                        d16 = dst_v[j, pl.ds(pl.multiple_of(t * 16, 16), 16)]
                        for lidx in range(16):
                            d = d16[lidx]
                            r = lax.shift_right_logical(d, 7)
                            gi = lax.bitwise_and(lax.shift_right_logical(d, 4), 7)
                            l = lax.bitwise_and(d, 15)
                            sl = pl.ds(pl.multiple_of(gi * 16, 16), 16)
                            hist_v[r, sl] = hist_v[r, sl] + jnp.where(
                                iota16 == l, one16, zero16f)
                        return carry2

                    lax.fori_loop(0, K // 16, cgroup, 0)
            return carry

        lax.fori_loop(0, NCHUNK, body, 0)

        if with_cnt:
            pltpu.sync_copy(iota_hbm, iota_v)
            pltpu.sync_copy(hist_v, cnt_sh.at[iota_v.at[0]], add=True)
        plsc.subcore_barrier()

        pltpu.sync_copy(s_sh.at[pl.ds(w * RPT, RPT)],
                        out_hbm.at[pl.ds(c * NROWS + w * RPT, RPT)])
        if with_cnt:
            @pl.when(w == 0)
            def _():
                pltpu.sync_copy(cnt_sh, cnt_hbm.at[pl.ds(c * HB, HB)])

    return k


def _sc_segment_sum(fxcat, srcidx, dstidx, zrows, iota, with_cnt):
    res = _make_sc_kernel(with_cnt)(fxcat, srcidx, dstidx, zrows, iota)
    if with_cnt:
        return res
    return (res[0] if isinstance(res, (list, tuple)) else res), None


def _powmsg(xmsg, p):
    cl = jnp.clip(xmsg, 0.0, 100.0)
    gen = jnp.exp(p * jnp.log(jnp.maximum(cl, 1e-30)))
    return jnp.where(p == 1.0, cl, gen)


def _prep_body(p_ref, x_ref, fx_ref):
    p = p_ref[0, 0]
    msg = jax.nn.relu(x_ref[...]) + EPS
    fx = _powmsg(msg, p)
    fx_ref[0] = fx[:, :RW]
    fx_ref[1] = fx[:, RW:]


def _prep(p, x):
    return pl.pallas_call(
        _prep_body,
        grid=(GRID,),
        in_specs=[
            pl.BlockSpec((1, 1), lambda i: (0, 0)),
            pl.BlockSpec((BN, C), lambda i: (i, 0)),
        ],
        out_specs=pl.BlockSpec((2, BN, RW), lambda i: (0, i, 0)),
        out_shape=jax.ShapeDtypeStruct((2, N, RW), jnp.float32),
    )(p, x)


def _mid_body(p_ref, xin_ref, sa_ref, sb_ref, cnta_ref, cntb_ref, w1_ref, b1_ref,
              h1_ref, sum_ref, ssq_ref, *, first):
    i = pl.program_id(0)
    p = p_ref[0, 0]
    xin = xin_ref[...]
    if not first:
        xin = jax.nn.relu(xin) + EPS
    s = jnp.concatenate([sa_ref[0], sb_ref[0]], axis=1)
    agg = s / jnp.maximum(cnta_ref[...] + cntb_ref[...], 1.0)
    out = _powmsg(agg, 1.0 / p)
    nrm = jnp.sqrt(jnp.sum(out * out, axis=1, keepdims=True))
    out = out / jnp.maximum(nrm, 1e-12)
    xnrm = jnp.sqrt(jnp.sum(xin * xin, axis=1, keepdims=True))
    out = out * xnrm + xin
    h1 = lax.dot_general(out, w1_ref[...], (((1,), (0,)), ((), ())),
                         preferred_element_type=jnp.float32) + b1_ref[...]
    h1_ref[...] = h1

    @pl.when(i == 0)
    def _():
        sum_ref[...] = jnp.zeros_like(sum_ref)
        ssq_ref[...] = jnp.zeros_like(ssq_ref)

    sum_ref[...] += jnp.sum(h1, axis=0, keepdims=True)
    ssq_ref[...] += jnp.sum(h1 * h1, axis=0, keepdims=True)


def _mid(p, xin, s2, cnta, cntb, w1, b1, first):
    return pl.pallas_call(
        functools.partial(_mid_body, first=first),
        grid=(GRID,),
        in_specs=[
            pl.BlockSpec((1, 1), lambda i: (0, 0)),
            pl.BlockSpec((BN, C), lambda i: (i, 0)),
            pl.BlockSpec((1, BN, RW), lambda i: (0, i, 0)),
            pl.BlockSpec((1, BN, RW), lambda i: (1, i, 0)),
            pl.BlockSpec((BN, 1), lambda i: (i, 0)),
            pl.BlockSpec((BN, 1), lambda i: (i, 0)),
            pl.BlockSpec((C, C), lambda i: (0, 0)),
            pl.BlockSpec((1, C), lambda i: (0, 0)),
        ],
        out_specs=[
            pl.BlockSpec((BN, C), lambda i: (i, 0)),
            pl.BlockSpec((1, C), lambda i: (0, 0)),
            pl.BlockSpec((1, C), lambda i: (0, 0)),
        ],
        out_shape=[
            jax.ShapeDtypeStruct((N, C), jnp.float32),
            jax.ShapeDtypeStruct((1, C), jnp.float32),
            jax.ShapeDtypeStruct((1, C), jnp.float32),
        ],
    )(p, xin, s2, s2, cnta, cntb, w1, b1)


def _bn_relu(h1, sum_, ssq, g, be):
    mu = sum_ * (1.0 / N)
    var = ssq * (1.0 / N) - mu * mu
    inv = lax.rsqrt(var + 1e-05)
    return jax.nn.relu((h1 - mu) * inv * g + be)


def _post_prep_body(h1_ref, sum_ref, ssq_ref, g_ref, be_ref, w2_ref, b2_ref,
                    pn_ref, c0_ref, fx_ref):
    h = _bn_relu(h1_ref[...], sum_ref[...], ssq_ref[...], g_ref[...], be_ref[...])
    c0 = lax.dot_general(h, w2_ref[...], (((1,), (0,)), ((), ())),
                         preferred_element_type=jnp.float32) + b2_ref[...]
    c0_ref[...] = c0
    pn = pn_ref[0, 0]
    msg = jax.nn.relu(c0) + 2.0 * EPS
    fx = _powmsg(msg, pn)
    fx_ref[0] = fx[:, :RW]
    fx_ref[1] = fx[:, RW:]


def _post_prep(h1, sum_, ssq, g, be, w2, b2, pn):
    return pl.pallas_call(
        _post_prep_body,
        grid=(GRID,),
        in_specs=[
            pl.BlockSpec((BN, C), lambda i: (i, 0)),
            pl.BlockSpec((1, C), lambda i: (0, 0)),
            pl.BlockSpec((1, C), lambda i: (0, 0)),
            pl.BlockSpec((1, C), lambda i: (0, 0)),
            pl.BlockSpec((1, C), lambda i: (0, 0)),
            pl.BlockSpec((C, C), lambda i: (0, 0)),
            pl.BlockSpec((1, C), lambda i: (0, 0)),
            pl.BlockSpec((1, 1), lambda i: (0, 0)),
        ],
        out_specs=[
            pl.BlockSpec((BN, C), lambda i: (i, 0)),
            pl.BlockSpec((2, BN, RW), lambda i: (0, i, 0)),
        ],
        out_shape=[
            jax.ShapeDtypeStruct((N, C), jnp.float32),
            jax.ShapeDtypeStruct((2, N, RW), jnp.float32),
        ],
    )(h1, sum_, ssq, g, be, w2, b2, pn)


def _post_final_body(h1_ref, sum_ref, ssq_ref, g_ref, be_ref, w2_ref, b2_ref,
                     h0_ref, we_ref, bexp_ref, y_ref):
    h = _bn_relu(h1_ref[...], sum_ref[...], ssq_ref[...], g_ref[...], be_ref[...])
    c1 = lax.dot_general(h, w2_ref[...], (((1,), (0,)), ((), ())),
                         preferred_element_type=jnp.float32) + b2_ref[...]
    t = h0_ref[...] + c1
    y = lax.dot_general(t, we_ref[...], (((1,), (0,)), ((), ())),
                        preferred_element_type=jnp.float32) + bexp_ref[...]
    y_ref[...] = jax.nn.relu(y) + EPS


def _post_final(h1, sum_, ssq, g, be, w2, b2, h0, we, bexp):
    return pl.pallas_call(
        _post_final_body,
        grid=(GRID,),
        in_specs=[
            pl.BlockSpec((BN, C), lambda i: (i, 0)),
            pl.BlockSpec((1, C), lambda i: (0, 0)),
            pl.BlockSpec((1, C), lambda i: (0, 0)),
            pl.BlockSpec((1, C), lambda i: (0, 0)),
            pl.BlockSpec((1, C), lambda i: (0, 0)),
            pl.BlockSpec((C, C), lambda i: (0, 0)),
            pl.BlockSpec((1, C), lambda i: (0, 0)),
            pl.BlockSpec((BN, C), lambda i: (i, 0)),
            pl.BlockSpec((C, 2 * C), lambda i: (0, 0)),
            pl.BlockSpec((1, 2 * C), lambda i: (0, 0)),
        ],
        out_specs=pl.BlockSpec((BN, 2 * C), lambda i: (i, 0)),
        out_shape=jax.ShapeDtypeStruct((N, 2 * C), jnp.float32),
    )(h1, sum_, ssq, g, be, w2, b2, h0, we, bexp)


def kernel(x, edge_index, p0, W1_0, b1_0, g_0, be_0, W2_0, b2_0,
           p1, W1_1, b1_1, g_1, be_1, W2_1, b2_1, We, bexp):
    src = edge_index[0]
    dst = edge_index[1]
    pad = EP - E
    srcp = jnp.concatenate([src, jnp.zeros((pad,), jnp.int32)]).reshape(NT, NCHUNK, K)
    dstp = jnp.concatenate([dst, jnp.full((pad,), N, jnp.int32)]).reshape(NT, NCHUNK, K)
    zrows = jnp.zeros((NROWS, RW), jnp.float32)
    iota = jnp.arange(HB, dtype=jnp.int32).reshape(1, HB)
    p0r = p0.reshape(1, 1)
    p1r = p1.reshape(1, 1)

    fx0 = _prep(p0r, x)
    s0, cnt2 = _sc_segment_sum(fx0.reshape(2 * N, RW), srcp, dstp, zrows,
                               iota, True)
    cnta = cnt2[:HB].reshape(HB * RW)[:N].reshape(N, 1)
    cntb = cnt2[HB:].reshape(HB * RW)[:N].reshape(N, 1)
    s0 = s0.reshape(2, NROWS, RW)
    h1_0, sm0, sq0 = _mid(p0r, x, s0, cnta, cntb, W1_0, b1_0.reshape(1, C),
                          first=True)
    c0, fx1 = _post_prep(h1_0, sm0, sq0, g_0.reshape(1, C), be_0.reshape(1, C),
                         W2_0, b2_0.reshape(1, C), p1r)
    s1, _ = _sc_segment_sum(fx1.reshape(2 * N, RW), srcp, dstp, zrows,
                            iota, False)
    s1 = s1.reshape(2, NROWS, RW)
    h1_1, sm1, sq1 = _mid(p1r, c0, s1, cnta, cntb, W1_1, b1_1.reshape(1, C),
                          first=False)
    return _post_final(h1_1, sm1, sq1, g_1.reshape(1, C), be_1.reshape(1, C),
                       W2_1, b2_1.reshape(1, C), x, We, bexp.reshape(1, 2 * C))


# speedup vs baseline: 1.2714x; 1.0657x over previous
import functools

import jax
import jax.numpy as jnp
from jax import lax
from jax.experimental import pallas as pl
from jax.experimental.pallas import tpu as pltpu
from jax.experimental.pallas import tpu_sc as plsc

N = 10000
E = 160000
C = 256
EPS = 1e-05

NT = 16
K = 128
NCHUNK = 79
NCPAD = 80
PCH = 40
EPT = NCHUNK * K
EP = NT * EPT
RW = 128
NROWS = 10112
RPT = NROWS // NT
HB = 80
BN = 2000
GRID = N // BN


@functools.lru_cache(maxsize=None)
def _make_sc_kernel(with_cnt):
    mesh = plsc.VectorSubcoreMesh(core_axis_name="c", subcore_axis_name="s")
    out_type = [jax.ShapeDtypeStruct((2 * NROWS, RW), jnp.float32)]
    if with_cnt:
        scratch = [
            pltpu.VMEM((NCPAD, K), jnp.int32),
            pltpu.VMEM((NCPAD, K), jnp.int32),
            pltpu.VMEM((K, RW), jnp.float32),
            pltpu.VMEM_SHARED((NROWS, RW), jnp.float32),
        ]
    else:
        scratch = [
            pltpu.VMEM((PCH, K), jnp.int32),
            pltpu.VMEM((PCH, K), jnp.int32),
            pltpu.VMEM((K, RW), jnp.float32),
            pltpu.VMEM((K, RW), jnp.float32),
            pltpu.VMEM_SHARED((NROWS, RW), jnp.float32),
            pltpu.SemaphoreType.DMA,
            pltpu.SemaphoreType.DMA,
            pltpu.SemaphoreType.DMA,
            pltpu.SemaphoreType.DMA,
        ]
    if with_cnt:
        out_type.append(jax.ShapeDtypeStruct((2 * HB, RW), jnp.float32))
        scratch += [
            pltpu.VMEM((HB, RW), jnp.float32),
            pltpu.VMEM_SHARED((HB, RW), jnp.float32),
            pltpu.VMEM((1, HB), jnp.int32),
            pltpu.SMEM((K,), jnp.int32),
        ]

    @functools.partial(pl.kernel, out_type=out_type, mesh=mesh,
                       scratch_types=scratch)
    def k(fx_hbm, src_hbm, dst_hbm, z_hbm, iota_hbm, *rest):
        if with_cnt:
            (out_hbm, cnt_hbm, src_v, dst_v, rows_v, s_sh,
             hist_v, cnt_sh, iota_v, dsm) = rest
        else:
            (out_hbm, src_v, dst_v, rows_a, rows_b, s_sh,
             gsa, gsb, ssa, ssb) = rest
        c = lax.axis_index("c")
        w = lax.axis_index("s")
        coff = c * N

        pltpu.sync_copy(z_hbm.at[pl.ds(w * RPT, RPT)], s_sh.at[pl.ds(w * RPT, RPT)])
        if with_cnt:
            @pl.when(w == 0)
            def _():
                pltpu.sync_copy(z_hbm.at[pl.ds(0, HB)], cnt_sh)

            zero16 = jnp.zeros((16,), jnp.float32)

            def zhist(r, carry):
                for t in range(RW // 16):
                    hist_v[r, pl.ds(t * 16, 16)] = zero16
                return carry

            lax.fori_loop(0, HB, zhist, 0)
        plsc.subcore_barrier()

        def addoff(j, carry):
            for t in range(K // 16):
                sl = pl.ds(t * 16, 16)
                src_v[j, sl] = src_v[j, sl] + coff
            return carry

        iota16 = lax.iota(jnp.int32, 16)
        one16 = jnp.ones((16,), jnp.float32)
        zero16f = jnp.zeros((16,), jnp.float32)

        if not with_cnt:
            for ph in range(2):
                pc = PCH if ph == 0 else NCHUNK - PCH
                pltpu.sync_copy(src_hbm.at[w, pl.ds(ph * PCH, PCH)], src_v)
                pltpu.sync_copy(dst_hbm.at[w, pl.ds(ph * PCH, PCH)], dst_v)
                lax.fori_loop(0, PCH, addoff, 0)

                def gth(j, buf, sem):
                    return pltpu.async_copy(fx_hbm.at[src_v.at[j]], buf, sem)

                def gwait(j, buf, sem):
                    pltpu.make_async_copy(fx_hbm.at[src_v.at[j]], buf, sem).wait()

                def sct(j, buf, sem):
                    return pltpu.async_copy(buf, s_sh.at[dst_v.at[j]], sem,
                                            add=True)

                def swait(j, buf, sem):
                    pltpu.make_async_copy(buf, s_sh.at[dst_v.at[j]], sem).wait()

                gth(0, rows_a, gsa)
                gwait(0, rows_a, gsa)
                sct(0, rows_a, ssa)
                gth(1, rows_b, gsb)

                def pair(jp, carry):
                    j0 = 2 * jp + 1
                    gwait(j0, rows_b, gsb)
                    sct(j0, rows_b, ssb)
                    swait(j0 - 1, rows_a, ssa)

                    @pl.when(j0 + 1 < pc)
                    def _():
                        gth(j0 + 1, rows_a, gsa)
                        gwait(j0 + 1, rows_a, gsa)
                        sct(j0 + 1, rows_a, ssa)

                    swait(j0, rows_b, ssb)

                    @pl.when(j0 + 2 < pc)
                    def _():
                        gth(j0 + 2, rows_b, gsb)
                    return carry

                lax.fori_loop(0, (pc - 1) // 2, pair, 0)
                if pc % 2 == 0:
                    gwait(pc - 1, rows_b, gsb)
                    sct(pc - 1, rows_b, ssb)
                    swait(pc - 2, rows_a, ssa)
                    swait(pc - 1, rows_b, ssb)
                else:
                    swait(pc - 1, rows_a, ssa)
            plsc.subcore_barrier()

            pltpu.sync_copy(s_sh.at[pl.ds(w * RPT, RPT)],
                            out_hbm.at[pl.ds(c * NROWS + w * RPT, RPT)])
            return

        pltpu.sync_copy(src_hbm.at[w], src_v)
        pltpu.sync_copy(dst_hbm.at[w], dst_v)
        lax.fori_loop(0, NCHUNK, addoff, 0)

        def body(j, carry):
            pltpu.sync_copy(fx_hbm.at[src_v.at[j]], rows_v)
            pltpu.sync_copy(rows_v, s_sh.at[dst_v.at[j]], add=True)
            if with_cnt:
                @pl.when(lax.bitwise_and(j, 1) == c)
                def _():
                    def cgroup(t, carry2):
                        d16 = dst_v[j, pl.ds(pl.multiple_of(t * 16, 16), 16)]
                        for lidx in range(16):
                            d = d16[lidx]
                            r = lax.shift_right_logical(d, 7)
                            gi = lax.bitwise_and(lax.shift_right_logical(d, 4), 7)
                            l = lax.bitwise_and(d, 15)
                            sl = pl.ds(pl.multiple_of(gi * 16, 16), 16)
                            hist_v[r, sl] = hist_v[r, sl] + jnp.where(
                                iota16 == l, one16, zero16f)
                        return carry2

                    lax.fori_loop(0, K // 16, cgroup, 0)
            return carry

        lax.fori_loop(0, NCHUNK, body, 0)

        if with_cnt:
            pltpu.sync_copy(iota_hbm, iota_v)
            pltpu.sync_copy(hist_v, cnt_sh.at[iota_v.at[0]], add=True)
        plsc.subcore_barrier()

        pltpu.sync_copy(s_sh.at[pl.ds(w * RPT, RPT)],
                        out_hbm.at[pl.ds(c * NROWS + w * RPT, RPT)])
        if with_cnt:
            @pl.when(w == 0)
            def _():
                pltpu.sync_copy(cnt_sh, cnt_hbm.at[pl.ds(c * HB, HB)])

    return k


def _sc_segment_sum(fxcat, srcidx, dstidx, zrows, iota, with_cnt):
    res = _make_sc_kernel(with_cnt)(fxcat, srcidx, dstidx, zrows, iota)
    if with_cnt:
        return res
    return (res[0] if isinstance(res, (list, tuple)) else res), None


def _powmsg(xmsg, p):
    cl = jnp.clip(xmsg, 0.0, 100.0)
    gen = jnp.exp(p * jnp.log(jnp.maximum(cl, 1e-30)))
    return jnp.where(p == 1.0, cl, gen)


def _prep_body(p_ref, x_ref, fx_ref):
    p = p_ref[0, 0]
    msg = jax.nn.relu(x_ref[...]) + EPS
    fx = _powmsg(msg, p)
    fx_ref[0] = fx[:, :RW]
    fx_ref[1] = fx[:, RW:]


def _prep(p, x):
    return pl.pallas_call(
        _prep_body,
        grid=(GRID,),
        in_specs=[
            pl.BlockSpec((1, 1), lambda i: (0, 0)),
            pl.BlockSpec((BN, C), lambda i: (i, 0)),
        ],
        out_specs=pl.BlockSpec((2, BN, RW), lambda i: (0, i, 0)),
        out_shape=jax.ShapeDtypeStruct((2, N, RW), jnp.float32),
    )(p, x)


def _mid_body(p_ref, xin_ref, sa_ref, sb_ref, cnta_ref, cntb_ref, w1_ref, b1_ref,
              h1_ref, sum_ref, ssq_ref, *, first):
    i = pl.program_id(0)
    p = p_ref[0, 0]
    xin = xin_ref[...]
    if not first:
        xin = jax.nn.relu(xin) + EPS
    s = jnp.concatenate([sa_ref[0], sb_ref[0]], axis=1)
    agg = s / jnp.maximum(cnta_ref[...] + cntb_ref[...], 1.0)
    out = _powmsg(agg, 1.0 / p)
    nrm = jnp.sqrt(jnp.sum(out * out, axis=1, keepdims=True))
    out = out / jnp.maximum(nrm, 1e-12)
    xnrm = jnp.sqrt(jnp.sum(xin * xin, axis=1, keepdims=True))
    out = out * xnrm + xin
    h1 = lax.dot_general(out, w1_ref[...], (((1,), (0,)), ((), ())),
                         preferred_element_type=jnp.float32) + b1_ref[...]
    h1_ref[...] = h1

    @pl.when(i == 0)
    def _():
        sum_ref[...] = jnp.zeros_like(sum_ref)
        ssq_ref[...] = jnp.zeros_like(ssq_ref)

    sum_ref[...] += jnp.sum(h1, axis=0, keepdims=True)
    ssq_ref[...] += jnp.sum(h1 * h1, axis=0, keepdims=True)


def _mid(p, xin, s2, cnta, cntb, w1, b1, first):
    return pl.pallas_call(
        functools.partial(_mid_body, first=first),
        grid=(GRID,),
        in_specs=[
            pl.BlockSpec((1, 1), lambda i: (0, 0)),
            pl.BlockSpec((BN, C), lambda i: (i, 0)),
            pl.BlockSpec((1, BN, RW), lambda i: (0, i, 0)),
            pl.BlockSpec((1, BN, RW), lambda i: (1, i, 0)),
            pl.BlockSpec((BN, 1), lambda i: (i, 0)),
            pl.BlockSpec((BN, 1), lambda i: (i, 0)),
            pl.BlockSpec((C, C), lambda i: (0, 0)),
            pl.BlockSpec((1, C), lambda i: (0, 0)),
        ],
        out_specs=[
            pl.BlockSpec((BN, C), lambda i: (i, 0)),
            pl.BlockSpec((1, C), lambda i: (0, 0)),
            pl.BlockSpec((1, C), lambda i: (0, 0)),
        ],
        out_shape=[
            jax.ShapeDtypeStruct((N, C), jnp.float32),
            jax.ShapeDtypeStruct((1, C), jnp.float32),
            jax.ShapeDtypeStruct((1, C), jnp.float32),
        ],
    )(p, xin, s2, s2, cnta, cntb, w1, b1)


def _bn_relu(h1, sum_, ssq, g, be):
    mu = sum_ * (1.0 / N)
    var = ssq * (1.0 / N) - mu * mu
    inv = lax.rsqrt(var + 1e-05)
    return jax.nn.relu((h1 - mu) * inv * g + be)


def _post_prep_body(h1_ref, sum_ref, ssq_ref, g_ref, be_ref, w2_ref, b2_ref,
                    pn_ref, c0_ref, fx_ref):
    h = _bn_relu(h1_ref[...], sum_ref[...], ssq_ref[...], g_ref[...], be_ref[...])
    c0 = lax.dot_general(h, w2_ref[...], (((1,), (0,)), ((), ())),
                         preferred_element_type=jnp.float32) + b2_ref[...]
    c0_ref[...] = c0
    pn = pn_ref[0, 0]
    msg = jax.nn.relu(c0) + 2.0 * EPS
    fx = _powmsg(msg, pn)
    fx_ref[0] = fx[:, :RW]
    fx_ref[1] = fx[:, RW:]


def _post_prep(h1, sum_, ssq, g, be, w2, b2, pn):
    return pl.pallas_call(
        _post_prep_body,
        grid=(GRID,),
        in_specs=[
            pl.BlockSpec((BN, C), lambda i: (i, 0)),
            pl.BlockSpec((1, C), lambda i: (0, 0)),
            pl.BlockSpec((1, C), lambda i: (0, 0)),
            pl.BlockSpec((1, C), lambda i: (0, 0)),
            pl.BlockSpec((1, C), lambda i: (0, 0)),
            pl.BlockSpec((C, C), lambda i: (0, 0)),
            pl.BlockSpec((1, C), lambda i: (0, 0)),
            pl.BlockSpec((1, 1), lambda i: (0, 0)),
        ],
        out_specs=[
            pl.BlockSpec((BN, C), lambda i: (i, 0)),
            pl.BlockSpec((2, BN, RW), lambda i: (0, i, 0)),
        ],
        out_shape=[
            jax.ShapeDtypeStruct((N, C), jnp.float32),
            jax.ShapeDtypeStruct((2, N, RW), jnp.float32),
        ],
    )(h1, sum_, ssq, g, be, w2, b2, pn)


def _post_final_body(h1_ref, sum_ref, ssq_ref, g_ref, be_ref, w2_ref, b2_ref,
                     h0_ref, we_ref, bexp_ref, y_ref):
    h = _bn_relu(h1_ref[...], sum_ref[...], ssq_ref[...], g_ref[...], be_ref[...])
    c1 = lax.dot_general(h, w2_ref[...], (((1,), (0,)), ((), ())),
                         preferred_element_type=jnp.float32) + b2_ref[...]
    t = h0_ref[...] + c1
    y = lax.dot_general(t, we_ref[...], (((1,), (0,)), ((), ())),
                        preferred_element_type=jnp.float32) + bexp_ref[...]
    y_ref[...] = jax.nn.relu(y) + EPS


def _post_final(h1, sum_, ssq, g, be, w2, b2, h0, we, bexp):
    return pl.pallas_call(
        _post_final_body,
        grid=(GRID,),
        in_specs=[
            pl.BlockSpec((BN, C), lambda i: (i, 0)),
            pl.BlockSpec((1, C), lambda i: (0, 0)),
            pl.BlockSpec((1, C), lambda i: (0, 0)),
            pl.BlockSpec((1, C), lambda i: (0, 0)),
            pl.BlockSpec((1, C), lambda i: (0, 0)),
            pl.BlockSpec((C, C), lambda i: (0, 0)),
            pl.BlockSpec((1, C), lambda i: (0, 0)),
            pl.BlockSpec((BN, C), lambda i: (i, 0)),
            pl.BlockSpec((C, 2 * C), lambda i: (0, 0)),
            pl.BlockSpec((1, 2 * C), lambda i: (0, 0)),
        ],
        out_specs=pl.BlockSpec((BN, 2 * C), lambda i: (i, 0)),
        out_shape=jax.ShapeDtypeStruct((N, 2 * C), jnp.float32),
    )(h1, sum_, ssq, g, be, w2, b2, h0, we, bexp)


def kernel(x, edge_index, p0, W1_0, b1_0, g_0, be_0, W2_0, b2_0,
           p1, W1_1, b1_1, g_1, be_1, W2_1, b2_1, We, bexp):
    src = edge_index[0]
    dst = edge_index[1]
    pad = EP - E
    srcp = jnp.concatenate([src, jnp.zeros((pad,), jnp.int32)]).reshape(NT, NCHUNK, K)
    srcp = jnp.concatenate([srcp, jnp.zeros((NT, 1, K), jnp.int32)], axis=1)
    dstp = jnp.concatenate([dst, jnp.full((pad,), N, jnp.int32)]).reshape(NT, NCHUNK, K)
    dstp = jnp.concatenate([dstp, jnp.full((NT, 1, K), N, jnp.int32)], axis=1)
    zrows = jnp.zeros((NROWS, RW), jnp.float32)
    iota = jnp.arange(HB, dtype=jnp.int32).reshape(1, HB)
    p0r = p0.reshape(1, 1)
    p1r = p1.reshape(1, 1)

    fx0 = _prep(p0r, x)
    s0, cnt2 = _sc_segment_sum(fx0.reshape(2 * N, RW), srcp, dstp, zrows,
                               iota, True)
    cnta = cnt2[:HB].reshape(HB * RW)[:N].reshape(N, 1)
    cntb = cnt2[HB:].reshape(HB * RW)[:N].reshape(N, 1)
    s0 = s0.reshape(2, NROWS, RW)
    h1_0, sm0, sq0 = _mid(p0r, x, s0, cnta, cntb, W1_0, b1_0.reshape(1, C),
                          first=True)
    c0, fx1 = _post_prep(h1_0, sm0, sq0, g_0.reshape(1, C), be_0.reshape(1, C),
                         W2_0, b2_0.reshape(1, C), p1r)
    s1, _ = _sc_segment_sum(fx1.reshape(2 * N, RW), srcp, dstp, zrows,
                            iota, False)
    s1 = s1.reshape(2, NROWS, RW)
    h1_1, sm1, sq1 = _mid(p1r, c0, s1, cnta, cntb, W1_1, b1_1.reshape(1, C),
                          first=False)
    return _post_final(h1_1, sm1, sq1, g_1.reshape(1, C), be_1.reshape(1, C),
                       W2_1, b2_1.reshape(1, C), x, We, bexp.reshape(1, 2 * C))


# speedup vs baseline: 1.3783x; 1.0841x over previous
import functools

import jax
import jax.numpy as jnp
from jax import lax
from jax.experimental import pallas as pl
from jax.experimental.pallas import tpu as pltpu
from jax.experimental.pallas import tpu_sc as plsc

N = 10000
E = 160000
C = 256
EPS = 1e-05

NT = 16
K = 128
NCHUNK = 79
NCPAD = 96
PCH = 40
PCHC = 24
EPT = NCHUNK * K
EP = NT * EPT
RW = 128
NROWS = 10112
RPT = NROWS // NT
HB = 80
BN = 2000
GRID = N // BN


@functools.lru_cache(maxsize=None)
def _make_sc_kernel(with_cnt):
    mesh = plsc.VectorSubcoreMesh(core_axis_name="c", subcore_axis_name="s")
    out_type = [jax.ShapeDtypeStruct((2 * NROWS, RW), jnp.float32)]
    pch = PCHC if with_cnt else PCH
    scratch = [
        pltpu.VMEM((pch, K), jnp.int32),
        pltpu.VMEM((pch, K), jnp.int32),
        pltpu.VMEM((K, RW), jnp.float32),
        pltpu.VMEM((K, RW), jnp.float32),
        pltpu.VMEM_SHARED((NROWS, RW), jnp.float32),
        pltpu.SemaphoreType.DMA,
        pltpu.SemaphoreType.DMA,
        pltpu.SemaphoreType.DMA,
        pltpu.SemaphoreType.DMA,
    ]
    if with_cnt:
        out_type.append(jax.ShapeDtypeStruct((2 * HB, RW), jnp.float32))
        scratch += [
            pltpu.VMEM((HB, RW), jnp.float32),
            pltpu.VMEM_SHARED((HB, RW), jnp.float32),
            pltpu.VMEM((1, HB), jnp.int32),
            pltpu.SMEM((K,), jnp.int32),
        ]

    @functools.partial(pl.kernel, out_type=out_type, mesh=mesh,
                       scratch_types=scratch)
    def k(fx_hbm, src_hbm, dst_hbm, z_hbm, iota_hbm, *rest):
        if with_cnt:
            (out_hbm, cnt_hbm, src_v, dst_v, rows_a, rows_b, s_sh,
             gsa, gsb, ssa, ssb, hist_v, cnt_sh, iota_v, dsm) = rest
        else:
            (out_hbm, src_v, dst_v, rows_a, rows_b, s_sh,
             gsa, gsb, ssa, ssb) = rest
        c = lax.axis_index("c")
        w = lax.axis_index("s")
        coff = c * N

        pltpu.sync_copy(z_hbm.at[pl.ds(w * RPT, RPT)], s_sh.at[pl.ds(w * RPT, RPT)])
        if with_cnt:
            @pl.when(w == 0)
            def _():
                pltpu.sync_copy(z_hbm.at[pl.ds(0, HB)], cnt_sh)

            zero16 = jnp.zeros((16,), jnp.float32)

            def zhist(r, carry):
                for t in range(RW // 16):
                    hist_v[r, pl.ds(t * 16, 16)] = zero16
                return carry

            lax.fori_loop(0, HB, zhist, 0)
        plsc.subcore_barrier()

        def addoff(j, carry):
            for t in range(K // 16):
                sl = pl.ds(t * 16, 16)
                src_v[j, sl] = src_v[j, sl] + coff
            return carry

        iota16 = lax.iota(jnp.int32, 16)
        one16 = jnp.ones((16,), jnp.float32)
        zero16f = jnp.zeros((16,), jnp.float32)

        def count(j):
            def cgroup(t, carry2):
                d16 = dst_v[j, pl.ds(pl.multiple_of(t * 16, 16), 16)]
                for lidx in range(16):
                    d = d16[lidx]
                    r = lax.shift_right_logical(d, 7)
                    gi = lax.bitwise_and(lax.shift_right_logical(d, 4), 7)
                    l = lax.bitwise_and(d, 15)
                    sl = pl.ds(pl.multiple_of(gi * 16, 16), 16)
                    hist_v[r, sl] = hist_v[r, sl] + jnp.where(
                        iota16 == l, one16, zero16f)
                return carry2

            lax.fori_loop(0, K // 16, cgroup, 0)

        def maybe_count(j, even):
            if with_cnt:
                @pl.when(c == (0 if even else 1))
                def _():
                    count(j)

        nph = NCHUNK // pch + 1
        for ph in range(nph):
            pc = pch if ph < nph - 1 else NCHUNK - (nph - 1) * pch
            pltpu.sync_copy(src_hbm.at[w, pl.ds(ph * pch, pch)], src_v)
            pltpu.sync_copy(dst_hbm.at[w, pl.ds(ph * pch, pch)], dst_v)
            lax.fori_loop(0, pch, addoff, 0)

            def gth(j, buf, sem):
                return pltpu.async_copy(fx_hbm.at[src_v.at[j]], buf, sem)

            def gwait(j, buf, sem):
                pltpu.make_async_copy(fx_hbm.at[src_v.at[j]], buf, sem).wait()

            def sct(j, buf, sem):
                return pltpu.async_copy(buf, s_sh.at[dst_v.at[j]], sem,
                                        add=True)

            def swait(j, buf, sem):
                pltpu.make_async_copy(buf, s_sh.at[dst_v.at[j]], sem).wait()

            gth(0, rows_a, gsa)
            gwait(0, rows_a, gsa)
            sct(0, rows_a, ssa)
            gth(1, rows_b, gsb)
            maybe_count(0, even=True)

            def pair(jp, carry):
                j0 = 2 * jp + 1
                gwait(j0, rows_b, gsb)
                sct(j0, rows_b, ssb)
                maybe_count(j0, even=False)
                swait(j0 - 1, rows_a, ssa)

                @pl.when(j0 + 1 < pc)
                def _():
                    gth(j0 + 1, rows_a, gsa)
                    gwait(j0 + 1, rows_a, gsa)
                    sct(j0 + 1, rows_a, ssa)

                @pl.when(j0 + 1 < pc)
                def _():
                    maybe_count(j0 + 1, even=True)

                swait(j0, rows_b, ssb)

                @pl.when(j0 + 2 < pc)
                def _():
                    gth(j0 + 2, rows_b, gsb)
                return carry

            lax.fori_loop(0, (pc - 1) // 2, pair, 0)
            if pc % 2 == 0:
                gwait(pc - 1, rows_b, gsb)
                sct(pc - 1, rows_b, ssb)
                maybe_count(pc - 1, even=False)
                swait(pc - 2, rows_a, ssa)
                swait(pc - 1, rows_b, ssb)
            else:
                swait(pc - 1, rows_a, ssa)

        if with_cnt:
            pltpu.sync_copy(iota_hbm, iota_v)
            pltpu.sync_copy(hist_v, cnt_sh.at[iota_v.at[0]], add=True)
        plsc.subcore_barrier()

        pltpu.sync_copy(s_sh.at[pl.ds(w * RPT, RPT)],
                        out_hbm.at[pl.ds(c * NROWS + w * RPT, RPT)])
        if with_cnt:
            @pl.when(w == 0)
            def _():
                pltpu.sync_copy(cnt_sh, cnt_hbm.at[pl.ds(c * HB, HB)])

    return k

def _sc_segment_sum(fxcat, srcidx, dstidx, zrows, iota, with_cnt):
    res = _make_sc_kernel(with_cnt)(fxcat, srcidx, dstidx, zrows, iota)
    if with_cnt:
        return res
    return (res[0] if isinstance(res, (list, tuple)) else res), None


def _powmsg(xmsg, p):
    cl = jnp.clip(xmsg, 0.0, 100.0)
    gen = jnp.exp(p * jnp.log(jnp.maximum(cl, 1e-30)))
    return jnp.where(p == 1.0, cl, gen)


def _prep_body(p_ref, x_ref, fx_ref):
    p = p_ref[0, 0]
    msg = jax.nn.relu(x_ref[...]) + EPS
    fx = _powmsg(msg, p)
    fx_ref[0] = fx[:, :RW]
    fx_ref[1] = fx[:, RW:]


def _prep(p, x):
    return pl.pallas_call(
        _prep_body,
        grid=(GRID,),
        in_specs=[
            pl.BlockSpec((1, 1), lambda i: (0, 0)),
            pl.BlockSpec((BN, C), lambda i: (i, 0)),
        ],
        out_specs=pl.BlockSpec((2, BN, RW), lambda i: (0, i, 0)),
        out_shape=jax.ShapeDtypeStruct((2, N, RW), jnp.float32),
    )(p, x)


def _mid_body(p_ref, xin_ref, sa_ref, sb_ref, cnta_ref, cntb_ref, w1_ref, b1_ref,
              h1_ref, sum_ref, ssq_ref, *, first):
    i = pl.program_id(0)
    p = p_ref[0, 0]
    xin = xin_ref[...]
    if not first:
        xin = jax.nn.relu(xin) + EPS
    s = jnp.concatenate([sa_ref[0], sb_ref[0]], axis=1)
    agg = s / jnp.maximum(cnta_ref[...] + cntb_ref[...], 1.0)
    out = _powmsg(agg, 1.0 / p)
    nrm = jnp.sqrt(jnp.sum(out * out, axis=1, keepdims=True))
    out = out / jnp.maximum(nrm, 1e-12)
    xnrm = jnp.sqrt(jnp.sum(xin * xin, axis=1, keepdims=True))
    out = out * xnrm + xin
    h1 = lax.dot_general(out, w1_ref[...], (((1,), (0,)), ((), ())),
                         preferred_element_type=jnp.float32) + b1_ref[...]
    h1_ref[...] = h1

    @pl.when(i == 0)
    def _():
        sum_ref[...] = jnp.zeros_like(sum_ref)
        ssq_ref[...] = jnp.zeros_like(ssq_ref)

    sum_ref[...] += jnp.sum(h1, axis=0, keepdims=True)
    ssq_ref[...] += jnp.sum(h1 * h1, axis=0, keepdims=True)


def _mid(p, xin, s2, cnta, cntb, w1, b1, first):
    return pl.pallas_call(
        functools.partial(_mid_body, first=first),
        grid=(GRID,),
        in_specs=[
            pl.BlockSpec((1, 1), lambda i: (0, 0)),
            pl.BlockSpec((BN, C), lambda i: (i, 0)),
            pl.BlockSpec((1, BN, RW), lambda i: (0, i, 0)),
            pl.BlockSpec((1, BN, RW), lambda i: (1, i, 0)),
            pl.BlockSpec((BN, 1), lambda i: (i, 0)),
            pl.BlockSpec((BN, 1), lambda i: (i, 0)),
            pl.BlockSpec((C, C), lambda i: (0, 0)),
            pl.BlockSpec((1, C), lambda i: (0, 0)),
        ],
        out_specs=[
            pl.BlockSpec((BN, C), lambda i: (i, 0)),
            pl.BlockSpec((1, C), lambda i: (0, 0)),
            pl.BlockSpec((1, C), lambda i: (0, 0)),
        ],
        out_shape=[
            jax.ShapeDtypeStruct((N, C), jnp.float32),
            jax.ShapeDtypeStruct((1, C), jnp.float32),
            jax.ShapeDtypeStruct((1, C), jnp.float32),
        ],
    )(p, xin, s2, s2, cnta, cntb, w1, b1)


def _bn_relu(h1, sum_, ssq, g, be):
    mu = sum_ * (1.0 / N)
    var = ssq * (1.0 / N) - mu * mu
    inv = lax.rsqrt(var + 1e-05)
    return jax.nn.relu((h1 - mu) * inv * g + be)


def _post_prep_body(h1_ref, sum_ref, ssq_ref, g_ref, be_ref, w2_ref, b2_ref,
                    pn_ref, c0_ref, fx_ref):
    h = _bn_relu(h1_ref[...], sum_ref[...], ssq_ref[...], g_ref[...], be_ref[...])
    c0 = lax.dot_general(h, w2_ref[...], (((1,), (0,)), ((), ())),
                         preferred_element_type=jnp.float32) + b2_ref[...]
    c0_ref[...] = c0
    pn = pn_ref[0, 0]
    msg = jax.nn.relu(c0) + 2.0 * EPS
    fx = _powmsg(msg, pn)
    fx_ref[0] = fx[:, :RW]
    fx_ref[1] = fx[:, RW:]


def _post_prep(h1, sum_, ssq, g, be, w2, b2, pn):
    return pl.pallas_call(
        _post_prep_body,
        grid=(GRID,),
        in_specs=[
            pl.BlockSpec((BN, C), lambda i: (i, 0)),
            pl.BlockSpec((1, C), lambda i: (0, 0)),
            pl.BlockSpec((1, C), lambda i: (0, 0)),
            pl.BlockSpec((1, C), lambda i: (0, 0)),
            pl.BlockSpec((1, C), lambda i: (0, 0)),
            pl.BlockSpec((C, C), lambda i: (0, 0)),
            pl.BlockSpec((1, C), lambda i: (0, 0)),
            pl.BlockSpec((1, 1), lambda i: (0, 0)),
        ],
        out_specs=[
            pl.BlockSpec((BN, C), lambda i: (i, 0)),
            pl.BlockSpec((2, BN, RW), lambda i: (0, i, 0)),
        ],
        out_shape=[
            jax.ShapeDtypeStruct((N, C), jnp.float32),
            jax.ShapeDtypeStruct((2, N, RW), jnp.float32),
        ],
    )(h1, sum_, ssq, g, be, w2, b2, pn)


def _post_final_body(h1_ref, sum_ref, ssq_ref, g_ref, be_ref, w2_ref, b2_ref,
                     h0_ref, we_ref, bexp_ref, y_ref):
    h = _bn_relu(h1_ref[...], sum_ref[...], ssq_ref[...], g_ref[...], be_ref[...])
    c1 = lax.dot_general(h, w2_ref[...], (((1,), (0,)), ((), ())),
                         preferred_element_type=jnp.float32) + b2_ref[...]
    t = h0_ref[...] + c1
    y = lax.dot_general(t, we_ref[...], (((1,), (0,)), ((), ())),
                        preferred_element_type=jnp.float32) + bexp_ref[...]
    y_ref[...] = jax.nn.relu(y) + EPS


def _post_final(h1, sum_, ssq, g, be, w2, b2, h0, we, bexp):
    return pl.pallas_call(
        _post_final_body,
        grid=(GRID,),
        in_specs=[
            pl.BlockSpec((BN, C), lambda i: (i, 0)),
            pl.BlockSpec((1, C), lambda i: (0, 0)),
            pl.BlockSpec((1, C), lambda i: (0, 0)),
            pl.BlockSpec((1, C), lambda i: (0, 0)),
            pl.BlockSpec((1, C), lambda i: (0, 0)),
            pl.BlockSpec((C, C), lambda i: (0, 0)),
            pl.BlockSpec((1, C), lambda i: (0, 0)),
            pl.BlockSpec((BN, C), lambda i: (i, 0)),
            pl.BlockSpec((C, 2 * C), lambda i: (0, 0)),
            pl.BlockSpec((1, 2 * C), lambda i: (0, 0)),
        ],
        out_specs=pl.BlockSpec((BN, 2 * C), lambda i: (i, 0)),
        out_shape=jax.ShapeDtypeStruct((N, 2 * C), jnp.float32),
    )(h1, sum_, ssq, g, be, w2, b2, h0, we, bexp)


def kernel(x, edge_index, p0, W1_0, b1_0, g_0, be_0, W2_0, b2_0,
           p1, W1_1, b1_1, g_1, be_1, W2_1, b2_1, We, bexp):
    src = edge_index[0]
    dst = edge_index[1]
    pad = EP - E
    srcp = jnp.concatenate([src, jnp.zeros((pad,), jnp.int32)]).reshape(NT, NCHUNK, K)
    srcp = jnp.concatenate(
        [srcp, jnp.zeros((NT, NCPAD - NCHUNK, K), jnp.int32)], axis=1)
    dstp = jnp.concatenate([dst, jnp.full((pad,), N, jnp.int32)]).reshape(NT, NCHUNK, K)
    dstp = jnp.concatenate(
        [dstp, jnp.full((NT, NCPAD - NCHUNK, K), N, jnp.int32)], axis=1)
    zrows = jnp.zeros((NROWS, RW), jnp.float32)
    iota = jnp.arange(HB, dtype=jnp.int32).reshape(1, HB)
    p0r = p0.reshape(1, 1)
    p1r = p1.reshape(1, 1)

    fx0 = _prep(p0r, x)
    s0, cnt2 = _sc_segment_sum(fx0.reshape(2 * N, RW), srcp, dstp, zrows,
                               iota, True)
    cnta = cnt2[:HB].reshape(HB * RW)[:N].reshape(N, 1)
    cntb = cnt2[HB:].reshape(HB * RW)[:N].reshape(N, 1)
    s0 = s0.reshape(2, NROWS, RW)
    h1_0, sm0, sq0 = _mid(p0r, x, s0, cnta, cntb, W1_0, b1_0.reshape(1, C),
                          first=True)
    c0, fx1 = _post_prep(h1_0, sm0, sq0, g_0.reshape(1, C), be_0.reshape(1, C),
                         W2_0, b2_0.reshape(1, C), p1r)
    s1, _ = _sc_segment_sum(fx1.reshape(2 * N, RW), srcp, dstp, zrows,
                            iota, False)
    s1 = s1.reshape(2, NROWS, RW)
    h1_1, sm1, sq1 = _mid(p1r, c0, s1, cnta, cntb, W1_1, b1_1.reshape(1, C),
                          first=False)
    return _post_final(h1_1, sm1, sq1, g_1.reshape(1, C), be_1.reshape(1, C),
                       W2_1, b2_1.reshape(1, C), x, We, bexp.reshape(1, 2 * C))
